# baseline (speedup 1.0000x reference)
import functools

import jax
import jax.numpy as jnp
from jax import lax
from jax.experimental import pallas as pl
from jax.experimental.pallas import tpu as pltpu
from jax.experimental.pallas import tpu_sc as plsc

NNB = 10
NC, NS = 2, 16
LANES = 16
NW = NC * NS


def kernel(msgs, ts, counts):
    N, L, D = msgs.shape
    NB = LANES
    NBLK = N // NB
    ITERS = (NBLK + NW - 1) // NW

    ts_flat = ts.reshape(N * L)

    mesh = plsc.VectorSubcoreMesh(
        core_axis_name="c", subcore_axis_name="s",
        num_cores=NC, num_subcores=NS)

    @functools.partial(
        pl.kernel,
        out_type=(
            jax.ShapeDtypeStruct((N, NNB, D), jnp.float32),
            jax.ShapeDtypeStruct((N * NNB,), jnp.int32),
        ),
        mesh=mesh,
        compiler_params=pltpu.CompilerParams(
            needs_layout_passes=False, use_tc_tiling_on_sc=True),
        scratch_types=[
            pltpu.VMEM((NB,), jnp.int32),
            pltpu.VMEM((NB * L,), jnp.int32),
            pltpu.VMEM((NB * NNB,), jnp.int32),
            pltpu.VMEM((NB, 24, D), jnp.float32),
            pltpu.VMEM((NB, 16, D), jnp.float32),
            pltpu.SemaphoreType.DMA,
        ],
    )
    def sc_kernel(msgs_hbm, ts_hbm, counts_hbm, feats_out, ts_out,
                  counts_v, ts_v, tso_v, msg_v, stage, sem):
        wid = lax.axis_index("s") * NC + lax.axis_index("c")

        def block_body(i, carry):
            b = wid + i * NW

            @pl.when(b < NBLK)
            def _():
                node0 = b * NB
                cp = pltpu.async_copy(
                    msgs_hbm.at[pl.ds(node0, NB)],
                    msg_v.at[:, pl.ds(0, L), :], sem)
                pltpu.sync_copy(counts_hbm.at[pl.ds(node0, NB)], counts_v)
                pltpu.sync_copy(ts_hbm.at[pl.ds(node0 * L, NB * L)], ts_v)

                lane = lax.iota(jnp.int32, LANES)
                c16 = counts_v[pl.ds(0, LANES)]
                for j in range(NNB):
                    idx = c16 - NNB + j
                    valid = idx >= 0
                    idx_cl = jnp.maximum(idx, 0)
                    tsv = plsc.load_gather(ts_v, [lane * L + idx_cl])
                    tsv = jnp.where(valid, tsv,
                                    jnp.full((LANES,), -1, jnp.int32))
                    plsc.store_scatter(tso_v, [lane * NNB + j], tsv)

                cp.wait()

                zero16 = jnp.zeros((LANES,), jnp.float32)
                for t in range(NB):
                    c = c16[t]
                    z = jnp.maximum(NNB - c, 0)
                    s = jnp.maximum(c - NNB, 0)
                    for j in range(NNB):
                        src = jnp.clip(s + j - z, 0, L - 1)
                        keep = j >= z
                        for v in range(D // LANES):
                            data = msg_v[t, src, pl.ds(v * LANES, LANES)]
                            stage[t, j, pl.ds(v * LANES, LANES)] = (
                                jnp.where(keep, data, zero16))

                pltpu.sync_copy(stage.at[:, pl.ds(0, NNB), :],
                                feats_out.at[pl.ds(node0, NB)])
                pltpu.sync_copy(
                    tso_v, ts_out.at[pl.ds(node0 * NNB, NB * NNB)])

            return carry

        lax.fori_loop(0, ITERS, block_body, jnp.int32(0))

    feats, ts_o = sc_kernel(msgs, ts_flat, counts)
    return feats, ts_o.reshape(N, NNB)


# speedup vs baseline: 1.1835x; 1.1835x over previous
import functools

import jax
import jax.numpy as jnp
from jax import lax
from jax.experimental import pallas as pl
from jax.experimental.pallas import tpu as pltpu
from jax.experimental.pallas import tpu_sc as plsc

NNB = 10
NC, NS = 2, 16
LANES = 16
NW = NC * NS
NB = 8
CH = 1568


def kernel(msgs, ts, counts):
    N, L, D = msgs.shape
    LPAD = 24
    NBLK_FULL = CH // NB
    NBLK_LAST = (N - (NW - 1) * CH) // NB
    OUTER = NBLK_FULL // 2

    ts_flat = ts.reshape(N * L)

    mesh = plsc.VectorSubcoreMesh(
        core_axis_name="c", subcore_axis_name="s",
        num_cores=NC, num_subcores=NS)

    @functools.partial(
        pl.kernel,
        out_type=(
            jax.ShapeDtypeStruct((N, NNB, D), jnp.float32),
            jax.ShapeDtypeStruct((N * NNB,), jnp.int32),
        ),
        mesh=mesh,
        compiler_params=pltpu.CompilerParams(
            needs_layout_passes=False, use_tc_tiling_on_sc=True),
        scratch_types=[
            pltpu.VMEM((CH + LANES,), jnp.int32),
            pltpu.VMEM((NB * L,), jnp.int32),
            pltpu.VMEM((NB * L,), jnp.int32),
            pltpu.VMEM((NB * NNB,), jnp.int32),
            pltpu.VMEM((NB * NNB,), jnp.int32),
            pltpu.VMEM((NB, LPAD, D), jnp.float32),
            pltpu.VMEM((NB, LPAD, D), jnp.float32),
            pltpu.VMEM((NB, 16, D), jnp.float32),
            pltpu.VMEM((NB, 16, D), jnp.float32),
            pltpu.SemaphoreType.DMA,
            pltpu.SemaphoreType.DMA,
            pltpu.SemaphoreType.DMA,
            pltpu.SemaphoreType.DMA,
        ],
    )
    def sc_kernel(msgs_hbm, ts_hbm, counts_hbm, feats_out, ts_out,
                  counts_v, ts_va, ts_vb, tso_va, tso_vb,
                  msg_v0, msg_v1, stage0, stage1,
                  sem_in0, sem_in1, sem_out0, sem_out1):
        wid = lax.axis_index("s") * NC + lax.axis_index("c")
        chunk_base = wid * CH
        nblk = jnp.where(wid == NW - 1, NBLK_LAST, NBLK_FULL)

        base_c = jnp.minimum(chunk_base, N - CH)
        delta = chunk_base - base_c
        pltpu.sync_copy(counts_hbm.at[pl.ds(base_c, CH)],
                        counts_v.at[pl.ds(0, CH)])

        ts_bufs = (ts_va, ts_vb)
        tso_bufs = (tso_va, tso_vb)
        msg_bufs = (msg_v0, msg_v1)
        stages = (stage0, stage1)
        sems_in = (sem_in0, sem_in1)
        sems_out = (sem_out0, sem_out1)
        lane = lax.iota(jnp.int32, LANES)

        def in_copies(slot, bi):
            node0 = chunk_base + bi * NB
            return (
                pltpu.make_async_copy(
                    msgs_hbm.at[pl.ds(node0, NB)],
                    msg_bufs[slot].at[:, pl.ds(0, L), :], sems_in[slot]),
                pltpu.make_async_copy(
                    ts_hbm.at[pl.ds(node0 * L, NB * L)],
                    ts_bufs[slot], sems_in[slot]),
            )

        def out_copies(slot, bi):
            node0 = chunk_base + bi * NB
            return (
                pltpu.make_async_copy(
                    stages[slot].at[:, pl.ds(0, NNB), :],
                    feats_out.at[pl.ds(node0, NB)], sems_out[slot]),
                pltpu.make_async_copy(
                    tso_bufs[slot],
                    ts_out.at[pl.ds(node0 * NNB, NB * NNB)],
                    sems_out[slot]),
            )

        def phase(slot, bi):
            @pl.when(bi < nblk)
            def _():
                for c in in_copies(slot, bi):
                    c.wait()

                for g in range(NB * NNB // LANES):
                    o = g * LANES + lane
                    t_vec = o // NNB
                    j_vec = o - t_vec * NNB
                    cv = plsc.load_gather(
                        counts_v, [delta + bi * NB + t_vec])
                    idx = cv - NNB + j_vec
                    idx_cl = jnp.maximum(idx, 0)
                    tsg = plsc.load_gather(
                        ts_bufs[slot], [t_vec * L + idx_cl])
                    tso_bufs[slot][pl.ds(g * LANES, LANES)] = jnp.where(
                        idx >= 0, tsg, jnp.full((LANES,), -1, jnp.int32))

                @pl.when(bi >= 2)
                def _():
                    for c in out_copies(slot, bi - 2):
                        c.wait()

                c16 = plsc.load_gather(
                    counts_v, [delta + bi * NB + lane])
                zero16 = jnp.zeros((LANES,), jnp.float32)
                for t in range(NB):
                    cnt = c16[t]
                    s = jnp.maximum(cnt - NNB, 0)
                    z = jnp.maximum(NNB - cnt, 0)
                    for j in range(NNB):
                        src = jnp.clip(s + j - z, 0, L - 1)
                        keep = j >= z
                        for v in range(D // LANES):
                            data = msg_bufs[slot][t, src,
                                                  pl.ds(v * LANES, LANES)]
                            stages[slot][t, j, pl.ds(v * LANES, LANES)] = (
                                jnp.where(keep, data, zero16))

                for c in out_copies(slot, bi):
                    c.start()

                @pl.when(bi + 2 < nblk)
                def _():
                    for c in in_copies(slot, bi + 2):
                        c.start()

        for c in in_copies(0, 0):
            c.start()
        for c in in_copies(1, 1):
            c.start()

        def outer(i, carry):
            phase(0, 2 * i)
            phase(1, 2 * i + 1)
            return carry

        lax.fori_loop(0, OUTER, outer, jnp.int32(0))

        for c in out_copies(0, nblk - 2):
            c.wait()
        for c in out_copies(1, nblk - 1):
            c.wait()

    feats, ts_o = sc_kernel(msgs, ts_flat, counts)
    return feats, ts_o.reshape(N, NNB)


# speedup vs baseline: 1.2752x; 1.0775x over previous
import functools

import jax
import jax.numpy as jnp
from jax import lax
from jax.experimental import pallas as pl
from jax.experimental.pallas import tpu as pltpu
from jax.experimental.pallas import tpu_sc as plsc

NNB = 10
NC, NS = 2, 16
LANES = 16
NW = NC * NS
NB = 8
CH = 1568


def _feats_sc(msgs, counts):
    N, L, D = msgs.shape
    NBLK_FULL = CH // NB
    NBLK_LAST = (N - (NW - 1) * CH) // NB
    OUTER = NBLK_FULL // 2

    mesh = plsc.VectorSubcoreMesh(
        core_axis_name="c", subcore_axis_name="s",
        num_cores=NC, num_subcores=NS)

    @functools.partial(
        pl.kernel,
        out_type=jax.ShapeDtypeStruct((N, NNB, D), jnp.float32),
        mesh=mesh,
        compiler_params=pltpu.CompilerParams(
            needs_layout_passes=False, use_tc_tiling_on_sc=True),
        scratch_types=[
            pltpu.VMEM((CH + LANES,), jnp.int32),
            pltpu.VMEM((NB, 24, D), jnp.float32),
            pltpu.VMEM((NB, 24, D), jnp.float32),
            pltpu.VMEM((NB, 16, D), jnp.float32),
            pltpu.VMEM((NB, 16, D), jnp.float32),
            pltpu.SemaphoreType.DMA,
            pltpu.SemaphoreType.DMA,
            pltpu.SemaphoreType.DMA,
            pltpu.SemaphoreType.DMA,
        ],
    )
    def sc_kernel(msgs_hbm, counts_hbm, feats_out,
                  counts_v, msg_v0, msg_v1, stage0, stage1,
                  sem_in0, sem_in1, sem_out0, sem_out1):
        wid = lax.axis_index("s") * NC + lax.axis_index("c")
        chunk_base = wid * CH
        nblk = jnp.where(wid == NW - 1, NBLK_LAST, NBLK_FULL)

        base_c = jnp.minimum(chunk_base, N - CH)
        delta = chunk_base - base_c
        pltpu.sync_copy(counts_hbm.at[pl.ds(base_c, CH)],
                        counts_v.at[pl.ds(0, CH)])

        msg_bufs = (msg_v0, msg_v1)
        stages = (stage0, stage1)
        sems_in = (sem_in0, sem_in1)
        sems_out = (sem_out0, sem_out1)
        lane = lax.iota(jnp.int32, LANES)

        def in_copy(slot, bi):
            node0 = chunk_base + bi * NB
            return pltpu.make_async_copy(
                msgs_hbm.at[pl.ds(node0, NB)],
                msg_bufs[slot].at[:, pl.ds(0, L), :], sems_in[slot])

        def out_copy(slot, bi):
            node0 = chunk_base + bi * NB
            return pltpu.make_async_copy(
                stages[slot].at[:, pl.ds(0, NNB), :],
                feats_out.at[pl.ds(node0, NB)], sems_out[slot])

        def phase(slot, bi):
            @pl.when(bi < nblk)
            def _():
                in_copy(slot, bi).wait()

                @pl.when(bi >= 2)
                def _():
                    out_copy(slot, bi - 2).wait()

                c16 = plsc.load_gather(
                    counts_v, [delta + bi * NB + lane])
                zero16 = jnp.zeros((LANES,), jnp.float32)
                for t in range(NB):
                    cnt = c16[t]
                    s = jnp.maximum(cnt - NNB, 0)
                    z = jnp.maximum(NNB - cnt, 0)
                    for j in range(NNB):
                        src = jnp.clip(s + j - z, 0, L - 1)
                        keep = j >= z
                        for v in range(D // LANES):
                            data = msg_bufs[slot][t, src,
                                                  pl.ds(v * LANES, LANES)]
                            stages[slot][t, j, pl.ds(v * LANES, LANES)] = (
                                jnp.where(keep, data, zero16))

                out_copy(slot, bi).start()

                @pl.when(bi + 2 < nblk)
                def _():
                    in_copy(slot, bi + 2).start()

        in_copy(0, 0).start()
        in_copy(1, 1).start()

        def outer(i, carry):
            phase(0, 2 * i)
            phase(1, 2 * i + 1)
            return carry

        lax.fori_loop(0, OUTER, outer, jnp.int32(0))

        out_copy(0, nblk - 2).wait()
        out_copy(1, nblk - 1).wait()

    return sc_kernel(msgs, counts)


def _ts_tc(ts, counts):
    N, L = ts.shape
    B = 400
    G = N // B
    counts3 = counts.reshape(G, 1, B)

    def tc_kernel(ts_ref, c_ref, out_ref):
        c = c_ref[0, 0, :][:, None]
        j = lax.broadcasted_iota(jnp.int32, (B, NNB), 1)
        idx = c - NNB + j
        acc = jnp.full((B, NNB), -1, jnp.int32)
        for l in range(L):
            tl = ts_ref[:, l][:, None]
            acc = jnp.where(idx == l, tl, acc)
        out_ref[:, :] = acc

    return pl.pallas_call(
        tc_kernel,
        grid=(G,),
        in_specs=[
            pl.BlockSpec((B, L), lambda i: (i, 0)),
            pl.BlockSpec((1, 1, B), lambda i: (i, 0, 0)),
        ],
        out_specs=pl.BlockSpec((B, NNB), lambda i: (i, 0)),
        out_shape=jax.ShapeDtypeStruct((N, NNB), jnp.int32),
    )(ts, counts3)


def kernel(msgs, ts, counts):
    return _feats_sc(msgs, counts), _ts_tc(ts, counts)


# speedup vs baseline: 2.2463x; 1.7615x over previous
import functools

import jax
import jax.numpy as jnp
from jax import lax
from jax.experimental import pallas as pl
from jax.experimental.pallas import tpu as pltpu
from jax.experimental.pallas import tpu_sc as plsc

NNB = 10
NC, NS = 2, 16
LANES = 16
NW = NC * NS
NB = 8
CH = 1568


def _feats_sc(msgs_t, counts):
    L, N, D = msgs_t.shape
    NBLK_FULL = CH // NB
    NBLK_LAST = (N - (NW - 1) * CH) // NB
    OUTER = NBLK_FULL // 2

    mesh = plsc.VectorSubcoreMesh(
        core_axis_name="c", subcore_axis_name="s",
        num_cores=NC, num_subcores=NS)

    @functools.partial(
        pl.kernel,
        out_type=jax.ShapeDtypeStruct((NNB, N, D), jnp.float32),
        mesh=mesh,
        compiler_params=pltpu.CompilerParams(
            needs_layout_passes=False, use_tc_tiling_on_sc=True),
        scratch_types=[
            pltpu.VMEM((CH + LANES,), jnp.int32),
            pltpu.VMEM((L, NB, D), jnp.float32),
            pltpu.VMEM((L, NB, D), jnp.float32),
            pltpu.VMEM((NNB, NB, D), jnp.float32),
            pltpu.VMEM((NNB, NB, D), jnp.float32),
            pltpu.SemaphoreType.DMA,
            pltpu.SemaphoreType.DMA,
            pltpu.SemaphoreType.DMA,
            pltpu.SemaphoreType.DMA,
        ],
    )
    def sc_kernel(msgs_hbm, counts_hbm, feats_out,
                  counts_v, msg_v0, msg_v1, stage0, stage1,
                  sem_in0, sem_in1, sem_out0, sem_out1):
        wid = lax.axis_index("s") * NC + lax.axis_index("c")
        chunk_base = wid * CH
        nblk = jnp.where(wid == NW - 1, NBLK_LAST, NBLK_FULL)

        base_c = jnp.minimum(chunk_base, N - CH)
        delta = chunk_base - base_c
        pltpu.sync_copy(counts_hbm.at[pl.ds(base_c, CH)],
                        counts_v.at[pl.ds(0, CH)])

        msg_bufs = (msg_v0, msg_v1)
        stages = (stage0, stage1)
        sems_in = (sem_in0, sem_in1)
        sems_out = (sem_out0, sem_out1)
        lane = lax.iota(jnp.int32, LANES)

        def in_copy(slot, bi):
            node0 = chunk_base + bi * NB
            return pltpu.make_async_copy(
                msgs_hbm.at[:, pl.ds(node0, NB), :],
                msg_bufs[slot], sems_in[slot])

        def out_copy(slot, bi):
            node0 = chunk_base + bi * NB
            return pltpu.make_async_copy(
                stages[slot],
                feats_out.at[:, pl.ds(node0, NB), :], sems_out[slot])

        def phase(slot, bi):
            @pl.when(bi < nblk)
            def _():
                in_copy(slot, bi).wait()

                @pl.when(bi >= 2)
                def _():
                    out_copy(slot, bi - 2).wait()

                c16 = plsc.load_gather(
                    counts_v, [delta + bi * NB + lane])
                zero16 = jnp.zeros((LANES,), jnp.float32)
                for t in range(NB):
                    cnt = c16[t]
                    s = jnp.maximum(cnt - NNB, 0)
                    z = jnp.maximum(NNB - cnt, 0)
                    for j in range(NNB):
                        src = jnp.clip(s + j - z, 0, L - 1)
                        keep = j >= z
                        for v in range(D // LANES):
                            data = msg_bufs[slot][src, t,
                                                  pl.ds(v * LANES, LANES)]
                            stages[slot][j, t, pl.ds(v * LANES, LANES)] = (
                                jnp.where(keep, data, zero16))

                out_copy(slot, bi).start()

                @pl.when(bi + 2 < nblk)
                def _():
                    in_copy(slot, bi + 2).start()

        in_copy(0, 0).start()
        in_copy(1, 1).start()

        def outer(i, carry):
            phase(0, 2 * i)
            phase(1, 2 * i + 1)
            return carry

        lax.fori_loop(0, OUTER, outer, jnp.int32(0))

        out_copy(0, nblk - 2).wait()
        out_copy(1, nblk - 1).wait()

    return sc_kernel(msgs_t, counts)


def _ts_tc(ts, counts):
    N, L = ts.shape
    B = 400
    G = N // B
    counts3 = counts.reshape(G, 1, B)

    def tc_kernel(ts_ref, c_ref, out_ref):
        c = c_ref[0, 0, :][:, None]
        j = lax.broadcasted_iota(jnp.int32, (B, NNB), 1)
        idx = c - NNB + j
        acc = jnp.full((B, NNB), -1, jnp.int32)
        for l in range(L):
            tl = ts_ref[:, l][:, None]
            acc = jnp.where(idx == l, tl, acc)
        out_ref[:, :] = acc

    return pl.pallas_call(
        tc_kernel,
        grid=(G,),
        in_specs=[
            pl.BlockSpec((B, L), lambda i: (i, 0)),
            pl.BlockSpec((1, 1, B), lambda i: (i, 0, 0)),
        ],
        out_specs=pl.BlockSpec((B, NNB), lambda i: (i, 0)),
        out_shape=jax.ShapeDtypeStruct((N, NNB), jnp.int32),
    )(ts, counts3)


def kernel(msgs, ts, counts):
    feats_t = _feats_sc(jnp.transpose(msgs, (1, 0, 2)), counts)
    return jnp.transpose(feats_t, (1, 0, 2)), _ts_tc(ts, counts)


# speedup vs baseline: 7.2939x; 3.2470x over previous
import functools

import jax
import jax.numpy as jnp
from jax import lax
from jax.experimental import pallas as pl
from jax.experimental.pallas import tpu as pltpu
from jax.experimental.pallas import tpu_sc as plsc

NNB = 10
NC, NS = 2, 16
LANES = 16
NW = NC * NS
NB = 16
CH = 1568
SROWS = 20


def _feats_sc(msgs_t, counts):
    L, N, D = msgs_t.shape
    NBLK_FULL = CH // NB
    NBLK_LAST = (N - (NW - 1) * CH) // NB
    OUTER = NBLK_FULL // 2

    mesh = plsc.VectorSubcoreMesh(
        core_axis_name="c", subcore_axis_name="s",
        num_cores=NC, num_subcores=NS)

    @functools.partial(
        pl.kernel,
        out_type=jax.ShapeDtypeStruct((NNB, N, D), jnp.float32),
        mesh=mesh,
        compiler_params=pltpu.CompilerParams(
            needs_layout_passes=False, use_tc_tiling_on_sc=True),
        scratch_types=[
            pltpu.VMEM((CH + LANES,), jnp.int32),
            pltpu.VMEM((SROWS, NB, D), jnp.float32),
            pltpu.VMEM((SROWS, NB, D), jnp.float32),
            pltpu.SemaphoreType.DMA,
            pltpu.SemaphoreType.DMA,
            pltpu.SemaphoreType.DMA,
            pltpu.SemaphoreType.DMA,
        ],
    )
    def sc_kernel(msgs_hbm, counts_hbm, feats_out,
                  counts_v, stage0, stage1,
                  sem_in0, sem_in1, sem_out0, sem_out1):
        wid = lax.axis_index("s") * NC + lax.axis_index("c")
        chunk_base = wid * CH
        nblk = jnp.where(wid == NW - 1, NBLK_LAST, NBLK_FULL)

        base_c = jnp.minimum(chunk_base, N - CH)
        delta = chunk_base - base_c
        pltpu.sync_copy(counts_hbm.at[pl.ds(base_c, CH)],
                        counts_v.at[pl.ds(0, CH)])

        stages = (stage0, stage1)
        sems_in = (sem_in0, sem_in1)
        sems_out = (sem_out0, sem_out1)
        lane = lax.iota(jnp.int32, LANES)
        zero16 = jnp.zeros((LANES,), jnp.float32)

        def in_drain(slot):
            pltpu.make_async_copy(
                msgs_hbm.at[pl.ds(0, NNB), pl.ds(0, NB), :],
                stages[slot].at[pl.ds(0, NNB), :, :],
                sems_in[slot]).wait()

        def out_copy(slot, bi):
            node0 = chunk_base + bi * NB
            return pltpu.make_async_copy(
                stages[slot].at[pl.ds(0, NNB), :, :],
                feats_out.at[:, pl.ds(node0, NB), :], sems_out[slot])

        def phase(slot, bi):
            other = 1 - slot

            @pl.when(bi < nblk)
            def _():
                node0 = chunk_base + bi * NB

                @pl.when(bi >= 2)
                def _():
                    out_copy(slot, bi - 2).wait()

                c16 = plsc.load_gather(
                    counts_v, [delta + bi * NB + lane])
                for t in range(NB):
                    cnt = c16[t]
                    s = jnp.maximum(cnt - NNB, 0)
                    z = jnp.maximum(NNB - cnt, 0)
                    pltpu.async_copy(
                        msgs_hbm.at[pl.ds(s, NNB), pl.ds(node0 + t, 1), :],
                        stages[slot].at[pl.ds(z, NNB), pl.ds(t, 1), :],
                        sems_in[slot])
                for t in range(NB):
                    z = jnp.maximum(NNB - c16[t], 0)

                    def zrow(j, carry, t=t):
                        for v in range(D // LANES):
                            stages[slot][j, t,
                                         pl.ds(v * LANES, LANES)] = zero16
                        return carry

                    lax.fori_loop(0, z, zrow, jnp.int32(0))

                @pl.when(bi >= 1)
                def _():
                    in_drain(other)
                    out_copy(other, bi - 1).start()

        def outer(i, carry):
            phase(0, 2 * i)
            phase(1, 2 * i + 1)
            return carry

        lax.fori_loop(0, OUTER, outer, jnp.int32(0))

        @pl.when(nblk % 2 == 0)
        def _():
            in_drain(1)
            out_copy(1, nblk - 1).start()

        @pl.when(nblk % 2 == 1)
        def _():
            in_drain(0)
            out_copy(0, nblk - 1).start()

        out_copy(0, 0).wait()
        out_copy(1, 1).wait()

    return sc_kernel(msgs_t, counts)


def _ts_tc(ts, counts):
    N, L = ts.shape
    B = 400
    G = N // B
    counts3 = counts.reshape(G, 1, B)

    def tc_kernel(ts_ref, c_ref, out_ref):
        c = c_ref[0, 0, :][:, None]
        j = lax.broadcasted_iota(jnp.int32, (B, NNB), 1)
        idx = c - NNB + j
        acc = jnp.full((B, NNB), -1, jnp.int32)
        for l in range(L):
            tl = ts_ref[:, l][:, None]
            acc = jnp.where(idx == l, tl, acc)
        out_ref[:, :] = acc

    return pl.pallas_call(
        tc_kernel,
        grid=(G,),
        in_specs=[
            pl.BlockSpec((B, L), lambda i: (i, 0)),
            pl.BlockSpec((1, 1, B), lambda i: (i, 0, 0)),
        ],
        out_specs=pl.BlockSpec((B, NNB), lambda i: (i, 0)),
        out_shape=jax.ShapeDtypeStruct((N, NNB), jnp.int32),
    )(ts, counts3)


def kernel(msgs, ts, counts):
    feats_t = _feats_sc(jnp.transpose(msgs, (1, 0, 2)), counts)
    return jnp.transpose(feats_t, (1, 0, 2)), _ts_tc(ts, counts)


# speedup vs baseline: 8.7134x; 1.1946x over previous
import functools

import jax
import jax.numpy as jnp
from jax import lax
from jax.experimental import pallas as pl
from jax.experimental.pallas import tpu as pltpu
from jax.experimental.pallas import tpu_sc as plsc

NNB = 10
NC, NS = 2, 16
LANES = 16
NW = NC * NS
NB = 16
CH = 1568
SROWS = 20


def _feats_sc(msgs_t, counts):
    L, N, D = msgs_t.shape
    NBLK_FULL = CH // NB
    NBLK_LAST = (N - (NW - 1) * CH) // NB
    OUTER = NBLK_FULL // 2

    mesh = plsc.VectorSubcoreMesh(
        core_axis_name="c", subcore_axis_name="s",
        num_cores=NC, num_subcores=NS)

    @functools.partial(
        pl.kernel,
        out_type=jax.ShapeDtypeStruct((NNB, N, D), jnp.float32),
        mesh=mesh,
        compiler_params=pltpu.CompilerParams(
            needs_layout_passes=False, use_tc_tiling_on_sc=True),
        scratch_types=[
            pltpu.VMEM((CH + LANES,), jnp.int32),
            pltpu.VMEM((SROWS, NB, D), jnp.float32),
            pltpu.VMEM((SROWS, NB, D), jnp.float32),
            pltpu.SemaphoreType.DMA,
            pltpu.SemaphoreType.DMA,
            pltpu.SemaphoreType.DMA,
            pltpu.SemaphoreType.DMA,
        ],
    )
    def sc_kernel(msgs_hbm, counts_hbm, feats_out,
                  counts_v, stage0, stage1,
                  sem_in0, sem_in1, sem_out0, sem_out1):
        wid = lax.axis_index("s") * NC + lax.axis_index("c")
        chunk_base = wid * CH
        nblk = jnp.where(wid == NW - 1, NBLK_LAST, NBLK_FULL)

        base_c = jnp.minimum(chunk_base, N - CH)
        delta = chunk_base - base_c
        pltpu.sync_copy(counts_hbm.at[pl.ds(base_c, CH)],
                        counts_v.at[pl.ds(0, CH)])

        stages = (stage0, stage1)
        sems_in = (sem_in0, sem_in1)
        sems_out = (sem_out0, sem_out1)
        lane = lax.iota(jnp.int32, LANES)
        zero16 = jnp.zeros((LANES,), jnp.float32)

        def in_drain(slot):
            pltpu.make_async_copy(
                msgs_hbm.at[pl.ds(0, NNB), pl.ds(0, NB), :],
                stages[slot].at[pl.ds(0, NNB), :, :],
                sems_in[slot]).wait()

        def out_copy(slot, bi):
            node0 = chunk_base + bi * NB
            return pltpu.make_async_copy(
                stages[slot].at[pl.ds(0, NNB), :, :],
                feats_out.at[:, pl.ds(node0, NB), :], sems_out[slot])

        def phase(slot, bi):
            other = 1 - slot

            @pl.when(bi < nblk)
            def _():
                node0 = chunk_base + bi * NB

                @pl.when(bi >= 2)
                def _():
                    out_copy(slot, bi - 2).wait()

                c16 = plsc.load_gather(
                    counts_v, [delta + bi * NB + lane])
                for t in range(NB):
                    cnt = c16[t]
                    s = jnp.maximum(cnt - NNB, 0)
                    z = jnp.maximum(NNB - cnt, 0)
                    pltpu.async_copy(
                        msgs_hbm.at[pl.ds(s, NNB), pl.ds(node0 + t, 1), :],
                        stages[slot].at[pl.ds(z, NNB), pl.ds(t, 1), :],
                        sems_in[slot])
                for t in range(NB):
                    z = jnp.maximum(NNB - c16[t], 0)

                    def zrow(j, carry, t=t):
                        for v in range(D // LANES):
                            stages[slot][j, t,
                                         pl.ds(v * LANES, LANES)] = zero16
                        return carry

                    lax.fori_loop(0, z, zrow, jnp.int32(0))

                @pl.when(bi >= 1)
                def _():
                    in_drain(other)
                    out_copy(other, bi - 1).start()

        def outer(i, carry):
            phase(0, 2 * i)
            phase(1, 2 * i + 1)
            return carry

        lax.fori_loop(0, OUTER, outer, jnp.int32(0))

        @pl.when(nblk % 2 == 0)
        def _():
            in_drain(1)
            out_copy(1, nblk - 1).start()

        @pl.when(nblk % 2 == 1)
        def _():
            in_drain(0)
            out_copy(0, nblk - 1).start()

        out_copy(0, 0).wait()
        out_copy(1, 1).wait()

    return sc_kernel(msgs_t, counts)


def _ts_tc(ts_t, counts):
    L, N = ts_t.shape
    B = 2048
    G = pl.cdiv(N, B)
    counts3 = counts.reshape(1, 1, N)

    def tc_kernel(ts_ref, c_ref, out_ref):
        c = c_ref[0, 0, :]
        idx = [c - NNB + j for j in range(NNB)]
        acc = [jnp.full((B,), -1, jnp.int32) for _ in range(NNB)]
        for l in range(L):
            tl = ts_ref[l, :]
            for j in range(NNB):
                acc[j] = jnp.where(idx[j] == l, tl, acc[j])
        for j in range(NNB):
            out_ref[j, :] = acc[j]

    return pl.pallas_call(
        tc_kernel,
        grid=(G,),
        in_specs=[
            pl.BlockSpec((L, B), lambda i: (0, i)),
            pl.BlockSpec((1, 1, B), lambda i: (0, 0, i)),
        ],
        out_specs=pl.BlockSpec((NNB, B), lambda i: (0, i)),
        out_shape=jax.ShapeDtypeStruct((NNB, N), jnp.int32),
    )(ts_t, counts3)


def kernel(msgs, ts, counts):
    feats_t = _feats_sc(jnp.transpose(msgs, (1, 0, 2)), counts)
    ts_o_t = _ts_tc(ts.T, counts)
    return jnp.transpose(feats_t, (1, 0, 2)), ts_o_t.T


# speedup vs baseline: 9.4002x; 1.0788x over previous
import functools

import jax
import jax.numpy as jnp
from jax import lax
from jax.experimental import pallas as pl
from jax.experimental.pallas import tpu as pltpu
from jax.experimental.pallas import tpu_sc as plsc

NNB = 10
NC, NS = 2, 16
LANES = 16
NW = NC * NS
NB = 16
CH = 1568
SROWS = 20


def _feats_sc(msgs_t, counts):
    L, N, D = msgs_t.shape
    NBLK_FULL = CH // NB
    NBLK_LAST = (N - (NW - 1) * CH) // NB
    OUTER = NBLK_FULL // 2

    mesh = plsc.VectorSubcoreMesh(
        core_axis_name="c", subcore_axis_name="s",
        num_cores=NC, num_subcores=NS)

    @functools.partial(
        pl.kernel,
        out_type=jax.ShapeDtypeStruct((NNB, N, D), jnp.float32),
        mesh=mesh,
        compiler_params=pltpu.CompilerParams(
            needs_layout_passes=False, use_tc_tiling_on_sc=True),
        scratch_types=[
            pltpu.VMEM((CH + LANES,), jnp.int32),
            pltpu.VMEM((SROWS, NB, D), jnp.float32),
            pltpu.VMEM((SROWS, NB, D), jnp.float32),
            pltpu.SemaphoreType.DMA,
            pltpu.SemaphoreType.DMA,
            pltpu.SemaphoreType.DMA,
            pltpu.SemaphoreType.DMA,
        ],
    )
    def sc_kernel(msgs_hbm, counts_hbm, feats_out,
                  counts_v, stage0, stage1,
                  sem_in0, sem_in1, sem_out0, sem_out1):
        wid = lax.axis_index("s") * NC + lax.axis_index("c")
        chunk_base = wid * CH
        nblk = jnp.where(wid == NW - 1, NBLK_LAST, NBLK_FULL)

        base_c = jnp.minimum(chunk_base, N - CH)
        delta = chunk_base - base_c
        pltpu.sync_copy(counts_hbm.at[pl.ds(base_c, CH)],
                        counts_v.at[pl.ds(0, CH)])

        stages = (stage0, stage1)
        sems_in = (sem_in0, sem_in1)
        sems_out = (sem_out0, sem_out1)
        lane = lax.iota(jnp.int32, LANES)
        zero16 = jnp.zeros((LANES,), jnp.float32)

        def in_drain(slot):
            pltpu.make_async_copy(
                msgs_hbm.at[pl.ds(0, NNB), pl.ds(0, NB), :],
                stages[slot].at[pl.ds(0, NNB), :, :],
                sems_in[slot]).wait()

        def out_copy(slot, bi):
            node0 = chunk_base + bi * NB
            return pltpu.make_async_copy(
                stages[slot].at[pl.ds(0, NNB), :, :],
                feats_out.at[:, pl.ds(node0, NB), :], sems_out[slot])

        def phase(slot, bi):
            other = 1 - slot

            @pl.when(bi < nblk)
            def _():
                node0 = chunk_base + bi * NB

                @pl.when(bi >= 2)
                def _():
                    out_copy(slot, bi - 2).wait()

                c16 = plsc.load_gather(
                    counts_v, [delta + bi * NB + lane])
                for t in range(NB):
                    cnt = c16[t]
                    s = jnp.maximum(cnt - NNB, 0)
                    z = jnp.maximum(NNB - cnt, 0)
                    pltpu.async_copy(
                        msgs_hbm.at[pl.ds(s, NNB), pl.ds(node0 + t, 1), :],
                        stages[slot].at[pl.ds(z, NNB), pl.ds(t, 1), :],
                        sems_in[slot])
                @pl.when(bi >= 1)
                def _():
                    in_drain(other)
                    out_copy(other, bi - 1).start()

                for t in range(NB):
                    z = jnp.maximum(NNB - c16[t], 0)

                    def zrow(j, carry, t=t):
                        for v in range(D // LANES):
                            stages[slot][j, t,
                                         pl.ds(v * LANES, LANES)] = zero16
                        return carry

                    lax.fori_loop(0, z, zrow, jnp.int32(0))

        def outer(i, carry):
            phase(0, 2 * i)
            phase(1, 2 * i + 1)
            return carry

        lax.fori_loop(0, OUTER, outer, jnp.int32(0))

        @pl.when(nblk % 2 == 0)
        def _():
            in_drain(1)
            out_copy(1, nblk - 1).start()

        @pl.when(nblk % 2 == 1)
        def _():
            in_drain(0)
            out_copy(0, nblk - 1).start()

        out_copy(0, 0).wait()
        out_copy(1, 1).wait()

    return sc_kernel(msgs_t, counts)


def _ts_tc(ts_t, counts):
    L, N = ts_t.shape
    B = 2048
    G = pl.cdiv(N, B)
    counts3 = counts.reshape(1, 1, N)

    def tc_kernel(ts_ref, c_ref, out_ref):
        c = c_ref[0, 0, :]
        idx = [c - NNB + j for j in range(NNB)]
        acc = [jnp.full((B,), -1, jnp.int32) for _ in range(NNB)]
        for l in range(L):
            tl = ts_ref[l, :]
            for j in range(NNB):
                acc[j] = jnp.where(idx[j] == l, tl, acc[j])
        for j in range(NNB):
            out_ref[j, :] = acc[j]

    return pl.pallas_call(
        tc_kernel,
        grid=(G,),
        in_specs=[
            pl.BlockSpec((L, B), lambda i: (0, i)),
            pl.BlockSpec((1, 1, B), lambda i: (0, 0, i)),
        ],
        out_specs=pl.BlockSpec((NNB, B), lambda i: (0, i)),
        out_shape=jax.ShapeDtypeStruct((NNB, N), jnp.int32),
    )(ts_t, counts3)


def kernel(msgs, ts, counts):
    feats_t = _feats_sc(jnp.transpose(msgs, (1, 0, 2)), counts)
    ts_o_t = _ts_tc(ts.T, counts)
    return jnp.transpose(feats_t, (1, 0, 2)), ts_o_t.T
